# Initial kernel scaffold; baseline (speedup 1.0000x reference)
import jax
import jax.numpy as jnp
from jax.experimental import pallas as pl


def kernel(x, edge_index, kern_id, W1, W2):
    raise NotImplementedError("write your pallas kernel here")


# speedup vs baseline: 13.3096x; 13.3096x over previous
import functools

import jax
import jax.numpy as jnp
from jax import lax
from jax.experimental import pallas as pl
from jax.experimental.pallas import tpu as pltpu
from jax.experimental.pallas import tpu_sc as plsc

N = 50000
E = 800000
C = 16
K = 27

NC = 2
NS = 16
NW = NC * NS
CHUNK = 128
CH = -(-E // (NW * CHUNK))
E_PAD = NW * CH * CHUNK
N_PAD = N + 48
RPT = N_PAD // NS
ROW_BLK = 2000
DST_PAD = N + 1


def _idx_body(src_ref, kid_ref, out_ref):
    out_ref[...] = src_ref[...] * K + kid_ref[...]


def _mm1_body(x_ref, w_ref, y_ref):
    y_ref[...] = jnp.dot(x_ref[...], w_ref[...],
                         preferred_element_type=jnp.float32)


def _mm_mid_body(p0_ref, p1_ref, w_ref, y_ref):
    h = jnp.maximum(p0_ref[0] + p1_ref[0], 0.0)
    y_ref[...] = jnp.dot(h, w_ref[...], preferred_element_type=jnp.float32)


def _final_body(p0_ref, p1_ref, o_ref):
    o_ref[...] = jnp.maximum(p0_ref[0] + p1_ref[0], 0.0)


def _flat_index(src_pad, kid_pad):
    rows = E_PAD // 128
    return pl.pallas_call(
        _idx_body,
        grid=(8,),
        in_specs=[
            pl.BlockSpec((rows // 8, 128), lambda i: (i, 0)),
            pl.BlockSpec((rows // 8, 128), lambda i: (i, 0)),
        ],
        out_specs=pl.BlockSpec((rows // 8, 128), lambda i: (i, 0)),
        out_shape=jax.ShapeDtypeStruct((rows, 128), jnp.int32),
    )(src_pad.reshape(rows, 128), kid_pad.reshape(rows, 128))


def _mm1(x, wcat):
    return pl.pallas_call(
        _mm1_body,
        grid=(N // ROW_BLK,),
        in_specs=[
            pl.BlockSpec((ROW_BLK, C), lambda i: (i, 0)),
            pl.BlockSpec((C, K * C), lambda i: (0, 0)),
        ],
        out_specs=pl.BlockSpec((ROW_BLK, K * C), lambda i: (i, 0)),
        out_shape=jax.ShapeDtypeStruct((N, K * C), jnp.float32),
    )(x, wcat)


def _mm_mid(partials, wcat):
    return pl.pallas_call(
        _mm_mid_body,
        grid=(N // ROW_BLK,),
        in_specs=[
            pl.BlockSpec((1, ROW_BLK, C), lambda i: (0, i, 0)),
            pl.BlockSpec((1, ROW_BLK, C), lambda i: (1, i, 0)),
            pl.BlockSpec((C, K * C), lambda i: (0, 0)),
        ],
        out_specs=pl.BlockSpec((ROW_BLK, K * C), lambda i: (i, 0)),
        out_shape=jax.ShapeDtypeStruct((N, K * C), jnp.float32),
    )(partials, partials, wcat)


def _final(partials):
    return pl.pallas_call(
        _final_body,
        grid=(N // ROW_BLK,),
        in_specs=[
            pl.BlockSpec((1, ROW_BLK, C), lambda i: (0, i, 0)),
            pl.BlockSpec((1, ROW_BLK, C), lambda i: (1, i, 0)),
        ],
        out_specs=pl.BlockSpec((ROW_BLK, C), lambda i: (i, 0)),
        out_shape=jax.ShapeDtypeStruct((N, C), jnp.float32),
    )(partials, partials)


@functools.cache
def _build_sc_scatter():
    mesh = plsc.VectorSubcoreMesh(core_axis_name="c", subcore_axis_name="s")

    @functools.partial(
        pl.kernel,
        out_type=jax.ShapeDtypeStruct((NC, N_PAD, C), jnp.float32),
        mesh=mesh,
        scratch_types=[
            pltpu.VMEM_SHARED((N_PAD, C), jnp.float32),
            pltpu.VMEM((CH, CHUNK), jnp.int32),
            pltpu.VMEM((CH, CHUNK), jnp.int32),
            pltpu.VMEM((CHUNK, C), jnp.float32),
            pltpu.SemaphoreType.DMA,
        ],
        compiler_params=pltpu.CompilerParams(use_tc_tiling_on_sc=False),
    )
    def sc_scatter(y_hbm, gidx_hbm, dst_hbm, zeros_hbm, out_hbm,
                   acc, gi_v, di_v, rows_v, sem):
        c = lax.axis_index("c")
        s = lax.axis_index("s")
        wid = c * NS + s
        r0 = s * RPT
        pltpu.sync_copy(zeros_hbm.at[pl.ds(r0, RPT)], acc.at[pl.ds(r0, RPT)])
        pltpu.sync_copy(gidx_hbm.at[wid], gi_v)
        pltpu.sync_copy(dst_hbm.at[wid], di_v)
        plsc.subcore_barrier()

        def body(j, carry):
            pltpu.async_copy(y_hbm.at[gi_v.at[j]], rows_v, sem).wait()
            pltpu.sync_copy(rows_v, acc.at[di_v.at[j]], add=True)
            return carry

        lax.fori_loop(0, CH, body, 0)
        plsc.subcore_barrier()
        pltpu.sync_copy(acc.at[pl.ds(r0, RPT)], out_hbm.at[c, pl.ds(r0, RPT)])

    return sc_scatter


def _sc_scatter(y, gidx, dst3, zeros):
    return _build_sc_scatter()(y, gidx, dst3, zeros)


def kernel(x, edge_index, kern_id, W1, W2):
    src = edge_index[0]
    dst = edge_index[1]
    src_pad = jnp.pad(src, (0, E_PAD - E))
    kid_pad = jnp.pad(kern_id, (0, E_PAD - E))
    dst_pad = jnp.pad(dst, (0, E_PAD - E), constant_values=DST_PAD)

    gidx = _flat_index(src_pad, kid_pad).reshape(NW, CH, CHUNK)
    dst3 = dst_pad.reshape(NW, CH, CHUNK)
    zeros = jnp.zeros((N_PAD, C), jnp.float32)

    w1cat = W1.transpose(1, 0, 2).reshape(C, K * C)
    w2cat = W2.transpose(1, 0, 2).reshape(C, K * C)

    y1 = _mm1(x, w1cat).reshape(N * K, C)
    p1 = _sc_scatter(y1, gidx, dst3, zeros)
    y2 = _mm_mid(p1, w2cat).reshape(N * K, C)
    p2 = _sc_scatter(y2, gidx, dst3, zeros)
    return _final(p2)


# speedup vs baseline: 13.4106x; 1.0076x over previous
import functools

import jax
import jax.numpy as jnp
from jax import lax
from jax.experimental import pallas as pl
from jax.experimental.pallas import tpu as pltpu
from jax.experimental.pallas import tpu_sc as plsc

N = 50000
E = 800000
C = 16
K = 27

NC = 2
NS = 16
NW = NC * NS
CHUNK = 128
GP = 4
GROUP_ROWS = GP * CHUNK
CH = 200
NG = CH // GP
E_PAD = NW * CH * CHUNK
N_PAD = N + 48
RPT = N_PAD // NS
ROW_BLK = 2000
DST_PAD = N + 1


def _idx_body(src_ref, kid_ref, out_ref):
    out_ref[...] = src_ref[...] * K + kid_ref[...]


def _mm1_body(x_ref, w_ref, y_ref):
    y_ref[...] = jnp.dot(x_ref[...], w_ref[...],
                         preferred_element_type=jnp.float32)


def _mm_mid_body(p0_ref, p1_ref, w_ref, y_ref):
    h = jnp.maximum(p0_ref[0] + p1_ref[0], 0.0)
    y_ref[...] = jnp.dot(h, w_ref[...], preferred_element_type=jnp.float32)


def _final_body(p0_ref, p1_ref, o_ref):
    o_ref[...] = jnp.maximum(p0_ref[0] + p1_ref[0], 0.0)


def _flat_index(src_pad, kid_pad):
    rows = E_PAD // 128
    return pl.pallas_call(
        _idx_body,
        grid=(8,),
        in_specs=[
            pl.BlockSpec((rows // 8, 128), lambda i: (i, 0)),
            pl.BlockSpec((rows // 8, 128), lambda i: (i, 0)),
        ],
        out_specs=pl.BlockSpec((rows // 8, 128), lambda i: (i, 0)),
        out_shape=jax.ShapeDtypeStruct((rows, 128), jnp.int32),
    )(src_pad.reshape(rows, 128), kid_pad.reshape(rows, 128))


def _mm1(x, wcat):
    return pl.pallas_call(
        _mm1_body,
        grid=(N // ROW_BLK,),
        in_specs=[
            pl.BlockSpec((ROW_BLK, C), lambda i: (i, 0)),
            pl.BlockSpec((C, K * C), lambda i: (0, 0)),
        ],
        out_specs=pl.BlockSpec((ROW_BLK, K * C), lambda i: (i, 0)),
        out_shape=jax.ShapeDtypeStruct((N, K * C), jnp.float32),
    )(x, wcat)


def _mm_mid(partials, wcat):
    return pl.pallas_call(
        _mm_mid_body,
        grid=(N // ROW_BLK,),
        in_specs=[
            pl.BlockSpec((1, ROW_BLK, C), lambda i: (0, i, 0)),
            pl.BlockSpec((1, ROW_BLK, C), lambda i: (1, i, 0)),
            pl.BlockSpec((C, K * C), lambda i: (0, 0)),
        ],
        out_specs=pl.BlockSpec((ROW_BLK, K * C), lambda i: (i, 0)),
        out_shape=jax.ShapeDtypeStruct((N, K * C), jnp.float32),
    )(partials, partials, wcat)


def _final(partials):
    return pl.pallas_call(
        _final_body,
        grid=(N // ROW_BLK,),
        in_specs=[
            pl.BlockSpec((1, ROW_BLK, C), lambda i: (0, i, 0)),
            pl.BlockSpec((1, ROW_BLK, C), lambda i: (1, i, 0)),
        ],
        out_specs=pl.BlockSpec((ROW_BLK, C), lambda i: (i, 0)),
        out_shape=jax.ShapeDtypeStruct((N, C), jnp.float32),
    )(partials, partials)


@functools.cache
def _build_sc_scatter():
    mesh = plsc.VectorSubcoreMesh(core_axis_name="c", subcore_axis_name="s")

    @functools.partial(
        pl.kernel,
        out_type=jax.ShapeDtypeStruct((NC, N_PAD, C), jnp.float32),
        mesh=mesh,
        scratch_types=[
            pltpu.VMEM_SHARED((N_PAD, C), jnp.float32),
            pltpu.VMEM((CH, CHUNK), jnp.int32),
            pltpu.VMEM((CH, CHUNK), jnp.int32),
            pltpu.VMEM((2, GROUP_ROWS, C), jnp.float32),
            pltpu.SemaphoreType.DMA,
            pltpu.SemaphoreType.DMA,
            pltpu.SemaphoreType.DMA,
            pltpu.SemaphoreType.DMA,
        ],
        compiler_params=pltpu.CompilerParams(use_tc_tiling_on_sc=False),
    )
    def sc_scatter(y_hbm, gidx_hbm, dst_hbm, zeros_hbm, out_hbm,
                   acc, gi_v, di_v, rows_v, gs0, gs1, ss0, ss1):
        c = lax.axis_index("c")
        s = lax.axis_index("s")
        wid = c * NS + s
        r0 = s * RPT
        pltpu.sync_copy(zeros_hbm.at[pl.ds(r0, RPT)], acc.at[pl.ds(r0, RPT)])
        pltpu.sync_copy(gidx_hbm.at[wid], gi_v)
        pltpu.sync_copy(dst_hbm.at[wid], di_v)
        plsc.subcore_barrier()

        def fire_g(g, bank, sem):
            for b in range(GP):
                pltpu.async_copy(y_hbm.at[gi_v.at[g * GP + b]],
                                 rows_v.at[bank, pl.ds(b * CHUNK, CHUNK)], sem)

        def fire_s(g, bank, sem):
            for b in range(GP):
                pltpu.async_copy(rows_v.at[bank, pl.ds(b * CHUNK, CHUNK)],
                                 acc.at[di_v.at[g * GP + b]], sem, add=True)

        def drain_g(sem):
            pltpu.make_async_copy(y_hbm.at[pl.ds(0, GROUP_ROWS)],
                                  rows_v.at[0], sem).wait()

        def drain_s(sem):
            pltpu.make_async_copy(rows_v.at[0],
                                  acc.at[pl.ds(0, GROUP_ROWS)], sem).wait()

        fire_g(0, 0, gs0)

        def body(i, carry):
            g0 = 2 * i
            drain_g(gs0)

            @pl.when(i > 0)
            def _():
                drain_s(ss1)

            fire_g(g0 + 1, 1, gs1)
            fire_s(g0, 0, ss0)
            drain_g(gs1)
            drain_s(ss0)

            @pl.when(i < NG // 2 - 1)
            def _():
                fire_g(g0 + 2, 0, gs0)

            fire_s(g0 + 1, 1, ss1)
            return carry

        lax.fori_loop(0, NG // 2, body, 0)
        drain_s(ss1)
        plsc.subcore_barrier()
        pltpu.sync_copy(acc.at[pl.ds(r0, RPT)], out_hbm.at[c, pl.ds(r0, RPT)])

    return sc_scatter


def _sc_scatter(y, gidx, dst3, zeros):
    return _build_sc_scatter()(y, gidx, dst3, zeros)


def kernel(x, edge_index, kern_id, W1, W2):
    src = edge_index[0]
    dst = edge_index[1]
    src_pad = jnp.pad(src, (0, E_PAD - E))
    kid_pad = jnp.pad(kern_id, (0, E_PAD - E))
    dst_pad = jnp.pad(dst, (0, E_PAD - E), constant_values=DST_PAD)

    gidx = _flat_index(src_pad, kid_pad).reshape(NW, CH, CHUNK)
    dst3 = dst_pad.reshape(NW, CH, CHUNK)
    zeros = jnp.zeros((N_PAD, C), jnp.float32)

    w1cat = W1.transpose(1, 0, 2).reshape(C, K * C)
    w2cat = W2.transpose(1, 0, 2).reshape(C, K * C)

    y1 = _mm1(x, w1cat).reshape(N * K, C)
    p1 = _sc_scatter(y1, gidx, dst3, zeros)
    y2 = _mm_mid(p1, w2cat).reshape(N * K, C)
    p2 = _sc_scatter(y2, gidx, dst3, zeros)
    return _final(p2)
